# baseline (speedup 1.0000x reference)
import functools

import jax
import jax.numpy as jnp
from jax import lax
from jax.experimental import pallas as pl
from jax.experimental.pallas import tpu as pltpu
from jax.experimental.pallas import tpu_sc as plsc

_B, _N, _C = 16, 20000, 85
_CONF = 0.25
_MAX_DET = 300
_CHUNK = 400
_NCHUNK = _N // _CHUNK
_VPC = _CHUNK // 16
_RPAD = 384
_NC = 2


def _sc_kernel(pred_hbm, out_hbm, seg_ref, rowidx_ref, rows_ref, sem):
    wid = lax.axis_index("s") * _NC + lax.axis_index("c")

    @pl.when(wid < _B)
    def _worker():
        b = wid
        iota = lax.iota(jnp.int32, 16)
        col4 = jnp.full((16,), 4, jnp.int32)

        def _init(k, carry):
            rowidx_ref[pl.ds(k * 16, 16)] = iota + (k * 16 + b * _N)
            return carry

        lax.fori_loop(0, (_RPAD + 16) // 16, _init, 0)

        def do_chunk(c, cnt):
            base_row = b * _N + c * _CHUNK
            pltpu.sync_copy(pred_hbm.at[pl.ds(base_row, _CHUNK)], seg_ref)

            def vec_body(j, cnt2):
                rows_local = iota + j * 16
                v = plsc.load_gather(seg_ref, [rows_local, col4])
                m = v > _CONF

                @pl.when(cnt2 < _MAX_DET)
                def _store():
                    plsc.store_compressed(
                        rowidx_ref.at[pl.ds(cnt2, 16)], rows_local + base_row, mask=m
                    )

                return cnt2 + jnp.sum(m.astype(jnp.int32))

            return lax.fori_loop(0, _VPC, vec_body, cnt)

        def seg_body(c, cnt):
            return lax.cond(
                cnt < _MAX_DET, lambda: do_chunk(c, cnt), lambda: cnt
            )

        cnt = lax.fori_loop(0, _NCHUNK, seg_body, 0)
        n_keep = jnp.minimum(cnt, _MAX_DET)

        def row_body(r, carry):
            row = rowidx_ref[pl.ds(r, 16)][0]
            pltpu.async_copy(
                pred_hbm.at[pl.ds(row, 1)], rows_ref.at[pl.ds(r, 1)], sem
            )
            return carry

        lax.fori_loop(0, _RPAD, row_body, 0)
        pltpu.make_async_copy(pred_hbm.at[pl.ds(0, _RPAD)], rows_ref, sem).wait()

        @pl.when(n_keep < _MAX_DET)
        def _zero_tail():
            def zero_row(r, carry):
                @pl.when(r >= n_keep)
                def _z():
                    for off in (0, 16, 32, 48, 64, _C - 16):
                        rows_ref[r, pl.ds(off, 16)] = jnp.zeros(
                            (16,), jnp.float32
                        )

                return carry

            lax.fori_loop(0, _MAX_DET, zero_row, 0)

        pltpu.sync_copy(rows_ref, out_hbm.at[b])


@jax.jit
def kernel(prediction):
    pred2 = prediction.reshape(_B * _N, _C)
    mesh = plsc.VectorSubcoreMesh(core_axis_name="c", subcore_axis_name="s")
    padded = pl.kernel(
        _sc_kernel,
        out_type=jax.ShapeDtypeStruct((_B, _RPAD, _C), jnp.float32),
        mesh=mesh,
        compiler_params=pltpu.CompilerParams(
            needs_layout_passes=False, use_tc_tiling_on_sc=False
        ),
        scratch_types=[
            pltpu.VMEM((_CHUNK, _C), jnp.float32),
            pltpu.VMEM((_RPAD + 16,), jnp.int32),
            pltpu.VMEM((_RPAD, _C), jnp.float32),
            pltpu.SemaphoreType.DMA,
        ],
    )(pred2)
    return padded[:, :_MAX_DET, :]


# speedup vs baseline: 1.5775x; 1.5775x over previous
import functools

import jax
import jax.numpy as jnp
from jax import lax
from jax.experimental import pallas as pl
from jax.experimental.pallas import tpu as pltpu
from jax.experimental.pallas import tpu_sc as plsc

_B, _N, _C = 16, 20000, 85
_CONF = 0.25
_MAX_DET = 300
_CHUNK = 400
_NCHUNK = _N // _CHUNK
_VPC = _CHUNK // 16
_RPAD = 384
_NC = 2


def _sc_kernel(pred_hbm, out_hbm, seg_ref, rowidx_ref, rows_ref, sem):
    wid = lax.axis_index("s") * _NC + lax.axis_index("c")

    @pl.when(wid < _B)
    def _worker():
        b = wid
        iota = lax.iota(jnp.int32, 16)
        col4 = jnp.full((16,), 4, jnp.int32)

        def _init(k, carry):
            rowidx_ref[pl.ds(k * 16, 16)] = iota + (k * 16 + b * _N)
            return carry

        lax.fori_loop(0, (_RPAD + 16) // 16, _init, 0)

        def do_chunk(c, cnt):
            base_row = b * _N + c * _CHUNK
            pltpu.sync_copy(pred_hbm.at[pl.ds(base_row, _CHUNK)], seg_ref)

            def vec_body(j, cnt2):
                rows_local = iota + j * 16
                v = plsc.load_gather(seg_ref, [rows_local, col4])
                m = v > _CONF

                @pl.when(cnt2 < _MAX_DET)
                def _store():
                    plsc.store_compressed(
                        rowidx_ref.at[pl.ds(cnt2, 16)], rows_local + base_row, mask=m
                    )

                return cnt2 + jnp.sum(m.astype(jnp.int32))

            return lax.fori_loop(0, _VPC, vec_body, cnt)

        def seg_body(c, cnt):
            return lax.cond(
                cnt < _MAX_DET, lambda: do_chunk(c, cnt), lambda: cnt
            )

        cnt = lax.fori_loop(0, _NCHUNK, seg_body, 0)
        n_keep = jnp.minimum(cnt, _MAX_DET)

        def row_body(r, carry):
            row = rowidx_ref[pl.ds(r, 16)][0]
            pltpu.async_copy(
                pred_hbm.at[pl.ds(row, 1)], rows_ref.at[pl.ds(r, 1)], sem
            )
            return carry

        lax.fori_loop(0, _RPAD, row_body, 0)
        pltpu.make_async_copy(pred_hbm.at[pl.ds(0, _RPAD)], rows_ref, sem).wait()

        @pl.when(n_keep < _MAX_DET)
        def _zero_tail():
            def zero_row(r, carry):
                @pl.when(r >= n_keep)
                def _z():
                    for off in (0, 16, 32, 48, 64, _C - 16):
                        rows_ref[r, pl.ds(off, 16)] = jnp.zeros(
                            (16,), jnp.float32
                        )

                return carry

            lax.fori_loop(0, _MAX_DET, zero_row, 0)

        pltpu.sync_copy(rows_ref, out_hbm.at[b])


@jax.jit
def kernel(prediction):
    pred2 = prediction.reshape(_B * _N, _C)
    mesh = plsc.VectorSubcoreMesh(core_axis_name="c", subcore_axis_name="s")
    padded = pl.kernel(
        _sc_kernel,
        out_type=jax.ShapeDtypeStruct((_B, _RPAD, _C), jnp.float32),
        mesh=mesh,
        compiler_params=pltpu.CompilerParams(
            needs_layout_passes=False, use_tc_tiling_on_sc=True
        ),
        scratch_types=[
            pltpu.VMEM((_CHUNK, _C), jnp.float32),
            pltpu.VMEM((_RPAD + 16,), jnp.int32),
            pltpu.VMEM((_RPAD, _C), jnp.float32),
            pltpu.SemaphoreType.DMA,
        ],
    )(pred2)
    return padded[:, :_MAX_DET, :]


# speedup vs baseline: 21.9664x; 13.9252x over previous
import functools

import jax
import jax.numpy as jnp
from jax import lax
from jax.experimental import pallas as pl
from jax.experimental.pallas import tpu as pltpu
from jax.experimental.pallas import tpu_sc as plsc

_B, _N, _C = 16, 20000, 85
_CONF = 0.25
_MAX_DET = 300
_OPAD = 304
_SEL = 320
_GRP = 2000
_NGRP = _N // _GRP
_VPG = _GRP // 16
_SPAN = 1024
_NC = 2
_CH0 = 43
_OBJ_ROW = 4 * _B


def _sc_kernel(pred_hbm, out_hbm, rowbuf_ref, rowidx_ref, spanall_ref,
               outall_ref, sem, sem2, osem):
    wid = lax.axis_index("s") * _NC + lax.axis_index("c")
    b = wid >> 1
    half = wid & 1
    ch_lo = half * _CH0
    n_ch = _CH0 - half
    iota = lax.iota(jnp.int32, 16)

    def _init(k, carry):
        rowidx_ref[pl.ds(k * 16, 16)] = jnp.zeros((16,), jnp.int32)
        return carry

    lax.fori_loop(0, _SEL // 16, _init, 0)

    pltpu.sync_copy(pred_hbm.at[pl.ds(_OBJ_ROW + b, 1)], rowbuf_ref)

    def do_group(t, cnt):
        base = t * _GRP

        def vec_body(j, cnt2):
            v = rowbuf_ref[0, pl.ds(base + j * 16, 16)]
            m = v > _CONF

            @pl.when(cnt2 < _MAX_DET)
            def _store():
                plsc.store_compressed(
                    rowidx_ref.at[pl.ds(cnt2, 16)], iota + (base + j * 16),
                    mask=m)

            return cnt2 + jnp.sum(m.astype(jnp.int32))

        return lax.fori_loop(0, _VPG, vec_body, cnt)

    def grp_body(t, cnt):
        return lax.cond(cnt < _MAX_DET, lambda: do_group(t, cnt), lambda: cnt)

    cnt = lax.fori_loop(0, _NGRP, grp_body, 0)
    n_keep = jnp.minimum(cnt, _MAX_DET)
    last = jnp.maximum(n_keep - 1, 0)
    idx_max = rowidx_ref[pl.ds(last, 16)][0]

    def fire(ci, carry):
        pltpu.async_copy(
            pred_hbm.at[pl.ds((ch_lo + ci) * _B + b, 1), pl.ds(0, _SPAN)],
            spanall_ref.at[pl.ds(ci, 1)], sem)
        return carry

    lax.fori_loop(0, n_ch, fire, 0)

    def drain(ci, carry):
        pltpu.make_async_copy(
            pred_hbm.at[pl.ds(0, 1), pl.ds(0, _SPAN)],
            spanall_ref.at[pl.ds(0, 1)], sem).wait()
        return carry

    lax.fori_loop(0, n_ch, drain, 0)

    zero16 = jnp.zeros((16,), jnp.int32)

    def chan_body(ci, carry):
        c_splat = lax.broadcast(ci, (16,))
        crow = (ch_lo + ci) * _B + b

        @pl.when(n_keep < _MAX_DET)
        def _zero_all():
            def z(u, carry2):
                outall_ref[ci, pl.ds(u * 16, 16)] = jnp.zeros(
                    (16,), jnp.float32)
                return carry2

            lax.fori_loop(0, _OPAD // 16, z, 0)

        @pl.when(n_keep >= _MAX_DET)
        def _zero_tail():
            outall_ref[ci, pl.ds(_OPAD - 16, 16)] = jnp.zeros(
                (16,), jnp.float32)

        def jv_main(u, carry2):
            idxv = rowidx_ref[pl.ds(u * 16, 16)]
            jvec = iota + u * 16
            m = jnp.logical_and(jvec < n_keep, idxv < _SPAN)
            g = plsc.load_gather(spanall_ref, [c_splat, idxv], mask=m)
            cur = outall_ref[ci, pl.ds(u * 16, 16)]
            outall_ref[ci, pl.ds(u * 16, 16)] = jnp.where(m, g, cur)
            return carry2

        lax.fori_loop(0, _OPAD // 16, jv_main, 0)

        @pl.when(idx_max >= _SPAN)
        def _full_row():
            pltpu.async_copy(
                pred_hbm.at[pl.ds(crow, 1)], rowbuf_ref, sem2).wait()

            def jv2(u, carry3):
                idxv = rowidx_ref[pl.ds(u * 16, 16)]
                jvec = iota + u * 16
                m = jnp.logical_and(jvec < n_keep, idxv >= _SPAN)
                g = plsc.load_gather(rowbuf_ref, [zero16, idxv], mask=m)
                cur = outall_ref[ci, pl.ds(u * 16, 16)]
                outall_ref[ci, pl.ds(u * 16, 16)] = jnp.where(m, g, cur)
                return carry3

            lax.fori_loop(0, _OPAD // 16, jv2, 0)

        pltpu.async_copy(
            outall_ref.at[pl.ds(ci, 1)], out_hbm.at[pl.ds(crow, 1)], osem)
        return carry

    lax.fori_loop(0, n_ch, chan_body, 0)

    def odrain(ci, carry):
        pltpu.make_async_copy(
            out_hbm.at[pl.ds(0, 1)], outall_ref.at[pl.ds(0, 1)], osem).wait()
        return carry

    lax.fori_loop(0, n_ch, odrain, 0)


@jax.jit
def kernel(prediction):
    pred_t = jnp.transpose(prediction, (2, 0, 1)).reshape(_C * _B, _N)
    mesh = plsc.VectorSubcoreMesh(core_axis_name="c", subcore_axis_name="s")
    out2d = pl.kernel(
        _sc_kernel,
        out_type=jax.ShapeDtypeStruct((_C * _B, _OPAD), jnp.float32),
        mesh=mesh,
        compiler_params=pltpu.CompilerParams(
            needs_layout_passes=False, use_tc_tiling_on_sc=True
        ),
        scratch_types=[
            pltpu.VMEM((1, _N), jnp.float32),
            pltpu.VMEM((_SEL,), jnp.int32),
            pltpu.VMEM((_CH0, _SPAN), jnp.float32),
            pltpu.VMEM((_CH0, _OPAD), jnp.float32),
            pltpu.SemaphoreType.DMA,
            pltpu.SemaphoreType.DMA,
            pltpu.SemaphoreType.DMA,
        ],
    )(pred_t)
    out_t = out2d.reshape(_C, _B, _OPAD)
    return jnp.transpose(out_t, (1, 2, 0))[:, :_MAX_DET, :]


# speedup vs baseline: 28.8109x; 1.3116x over previous
import functools

import jax
import jax.numpy as jnp
from jax import lax
from jax.experimental import pallas as pl
from jax.experimental.pallas import tpu as pltpu
from jax.experimental.pallas import tpu_sc as plsc

_B, _N, _C = 16, 20000, 85
_CONF = 0.25
_MAX_DET = 300
_OPAD = 304
_OV = _OPAD // 16
_SEL = 320
_GRP = 2048
_VPG = _GRP // 16
_SCAN0 = 4096
_NGRP = _N // _GRP
_TAIL0 = _NGRP * _GRP
_TAILV = (_N - _TAIL0) // 16
_SPAN = 1024
_NC = 2
_CH0 = 43
_OBJ_ROW = 4 * _B


def _sc_kernel(pred_hbm, out_hbm, rowbuf_ref, rowidx_ref, spanall_ref,
               outall_ref, sem, sem2, osem):
    wid = lax.axis_index("s") * _NC + lax.axis_index("c")
    b = wid >> 1
    half = wid & 1
    ch_lo = half * _CH0
    n_ch = _CH0 - half
    iota = lax.iota(jnp.int32, 16)

    pltpu.async_copy(
        pred_hbm.at[pl.ds(_OBJ_ROW + b, 1), pl.ds(0, _SCAN0)],
        rowbuf_ref.at[pl.ds(0, 1), pl.ds(0, _SCAN0)], sem2)

    def fire(ci, carry):
        pltpu.async_copy(
            pred_hbm.at[pl.ds((ch_lo + ci) * _B + b, 1), pl.ds(0, _SPAN)],
            spanall_ref.at[pl.ds(ci, 1)], sem)
        return carry

    lax.fori_loop(0, n_ch, fire, 0)

    pltpu.make_async_copy(
        pred_hbm.at[pl.ds(0, 1), pl.ds(0, _SCAN0)],
        rowbuf_ref.at[pl.ds(0, 1), pl.ds(0, _SCAN0)], sem2).wait()

    def _init(k, carry):
        rowidx_ref[pl.ds(k * 16, 16)] = jnp.zeros((16,), jnp.int32)
        return carry

    lax.fori_loop(0, _SEL // 16, _init, 0)

    def compact(base, nvec, cnt0):
        def vec_body(j, cnt2):
            v = rowbuf_ref[0, pl.ds(base + j * 16, 16)]
            m = v > _CONF

            @pl.when(cnt2 < _MAX_DET)
            def _store():
                plsc.store_compressed(
                    rowidx_ref.at[pl.ds(cnt2, 16)], iota + (base + j * 16),
                    mask=m)

            return cnt2 + jnp.sum(m.astype(jnp.int32))

        return lax.fori_loop(0, nvec, vec_body, cnt0)

    def grp(t, cnt):
        return lax.cond(
            cnt < _MAX_DET, lambda: compact(t * _GRP, _VPG, cnt), lambda: cnt)

    cnt = lax.fori_loop(0, _SCAN0 // _GRP, grp, 0)

    def scan_rest():
        pltpu.sync_copy(pred_hbm.at[pl.ds(_OBJ_ROW + b, 1)], rowbuf_ref)
        c2 = lax.fori_loop(_SCAN0 // _GRP, _NGRP, grp, cnt)
        return lax.cond(
            c2 < _MAX_DET, lambda: compact(_TAIL0, _TAILV, c2), lambda: c2)

    cnt = lax.cond(cnt < _MAX_DET, scan_rest, lambda: cnt)
    n_keep = jnp.minimum(cnt, _MAX_DET)
    last = jnp.maximum(n_keep - 1, 0)
    idx_max = rowidx_ref[pl.ds(last, 16)][0]

    def drain(ci, carry):
        pltpu.make_async_copy(
            pred_hbm.at[pl.ds(0, 1), pl.ds(0, _SPAN)],
            spanall_ref.at[pl.ds(0, 1)], sem).wait()
        return carry

    lax.fori_loop(0, n_ch, drain, 0)

    zero16 = jnp.zeros((16,), jnp.int32)
    zf16 = jnp.zeros((16,), jnp.float32)

    def u_body(u, carry):
        idxv = rowidx_ref[pl.ds(u * 16, 16)]
        jvec = iota + u * 16
        m = jnp.logical_and(jvec < n_keep, idxv < _SPAN)

        def ci_body(ci, carry2):
            g = plsc.load_gather(
                spanall_ref, [lax.broadcast(ci, (16,)), idxv], mask=m)
            outall_ref[ci, pl.ds(u * 16, 16)] = jnp.where(m, g, zf16)
            return carry2

        lax.fori_loop(0, n_ch, ci_body, 0)
        return carry

    lax.fori_loop(0, _OV, u_body, 0)

    @pl.when(idx_max >= _SPAN)
    def _full_rows():
        def ci_body(ci, carry):
            crow = (ch_lo + ci) * _B + b
            pltpu.async_copy(
                pred_hbm.at[pl.ds(crow, 1)], rowbuf_ref, sem2).wait()

            def jv2(u, carry2):
                idxv = rowidx_ref[pl.ds(u * 16, 16)]
                jvec = iota + u * 16
                m = jnp.logical_and(jvec < n_keep, idxv >= _SPAN)
                g = plsc.load_gather(rowbuf_ref, [zero16, idxv], mask=m)
                cur = outall_ref[ci, pl.ds(u * 16, 16)]
                outall_ref[ci, pl.ds(u * 16, 16)] = jnp.where(m, g, cur)
                return carry2

            lax.fori_loop(0, _OV, jv2, 0)
            return carry

        lax.fori_loop(0, n_ch, ci_body, 0)

    def wb(ci, carry):
        pltpu.async_copy(
            outall_ref.at[pl.ds(ci, 1)],
            out_hbm.at[pl.ds((ch_lo + ci) * _B + b, 1)], osem)
        return carry

    lax.fori_loop(0, n_ch, wb, 0)

    def odrain(ci, carry):
        pltpu.make_async_copy(
            out_hbm.at[pl.ds(0, 1)], outall_ref.at[pl.ds(0, 1)], osem).wait()
        return carry

    lax.fori_loop(0, n_ch, odrain, 0)


@jax.jit
def kernel(prediction):
    pred_t = jnp.transpose(prediction, (2, 0, 1)).reshape(_C * _B, _N)
    mesh = plsc.VectorSubcoreMesh(core_axis_name="c", subcore_axis_name="s")
    out2d = pl.kernel(
        _sc_kernel,
        out_type=jax.ShapeDtypeStruct((_C * _B, _OPAD), jnp.float32),
        mesh=mesh,
        compiler_params=pltpu.CompilerParams(
            needs_layout_passes=False, use_tc_tiling_on_sc=True
        ),
        scratch_types=[
            pltpu.VMEM((1, _N), jnp.float32),
            pltpu.VMEM((_SEL,), jnp.int32),
            pltpu.VMEM((_CH0, _SPAN), jnp.float32),
            pltpu.VMEM((_CH0, _OPAD), jnp.float32),
            pltpu.SemaphoreType.DMA,
            pltpu.SemaphoreType.DMA,
            pltpu.SemaphoreType.DMA,
        ],
    )(pred_t)
    out_t = out2d.reshape(_C, _B, _OPAD)
    return jnp.transpose(out_t, (1, 2, 0))[:, :_MAX_DET, :]


# speedup vs baseline: 29.1479x; 1.0117x over previous
import functools

import jax
import jax.numpy as jnp
from jax import lax
from jax.experimental import pallas as pl
from jax.experimental.pallas import tpu as pltpu
from jax.experimental.pallas import tpu_sc as plsc

_B, _N, _C = 16, 20000, 85
_CONF = 0.25
_MAX_DET = 300
_OPAD = 304
_OV = _OPAD // 16
_SEL = 320
_GRP = 1024
_VPG = _GRP // 16
_SCAN0 = 4096
_NGRP = _N // _GRP
_TAIL0 = _NGRP * _GRP
_TAILV = (_N - _TAIL0) // 16
_SPAN = 1024
_NC = 2
_CH0 = 43
_OBJ_ROW = 4 * _B


def _sc_kernel(pred_hbm, out_hbm, rowbuf_ref, rowidx_ref, spanall_ref,
               outall_ref, sem, sem2, osem):
    wid = lax.axis_index("s") * _NC + lax.axis_index("c")
    b = wid >> 1
    half = wid & 1
    ch_lo = half * _CH0
    n_ch = _CH0 - half
    iota = lax.iota(jnp.int32, 16)

    pltpu.async_copy(
        pred_hbm.at[pl.ds(_OBJ_ROW + b, 1), pl.ds(0, _SCAN0)],
        rowbuf_ref.at[pl.ds(0, 1), pl.ds(0, _SCAN0)], sem2)

    def fire(ci, carry):
        pltpu.async_copy(
            pred_hbm.at[pl.ds((ch_lo + ci) * _B + b, 1), pl.ds(0, _SPAN)],
            spanall_ref.at[pl.ds(ci, 1)], sem)
        return carry

    lax.fori_loop(0, n_ch, fire, 0)

    pltpu.make_async_copy(
        pred_hbm.at[pl.ds(0, 1), pl.ds(0, _SCAN0)],
        rowbuf_ref.at[pl.ds(0, 1), pl.ds(0, _SCAN0)], sem2).wait()

    def _init(k, carry):
        rowidx_ref[pl.ds(k * 16, 16)] = jnp.zeros((16,), jnp.int32)
        return carry

    lax.fori_loop(0, _SEL // 16, _init, 0)

    def compact(base, nvec, cnt0):
        def vec_body(j, cnt2):
            v = rowbuf_ref[0, pl.ds(base + j * 16, 16)]
            m = v > _CONF
            plsc.store_compressed(
                rowidx_ref.at[pl.ds(jnp.minimum(cnt2, _MAX_DET), 16)],
                iota + (base + j * 16), mask=m)
            return cnt2 + plsc.all_reduce_population_count(m)[0]

        return lax.fori_loop(0, nvec, vec_body, cnt0)

    def grp(t, cnt):
        return lax.cond(
            cnt < _MAX_DET, lambda: compact(t * _GRP, _VPG, cnt), lambda: cnt)

    cnt = lax.fori_loop(0, _SCAN0 // _GRP, grp, 0)

    def scan_rest():
        pltpu.sync_copy(pred_hbm.at[pl.ds(_OBJ_ROW + b, 1)], rowbuf_ref)
        c2 = lax.fori_loop(_SCAN0 // _GRP, _NGRP, grp, cnt)
        return lax.cond(
            c2 < _MAX_DET, lambda: compact(_TAIL0, _TAILV, c2), lambda: c2)

    cnt = lax.cond(cnt < _MAX_DET, scan_rest, lambda: cnt)
    n_keep = jnp.minimum(cnt, _MAX_DET)
    last = jnp.maximum(n_keep - 1, 0)
    idx_max = rowidx_ref[pl.ds(last, 16)][0]

    def drain(ci, carry):
        pltpu.make_async_copy(
            pred_hbm.at[pl.ds(0, 1), pl.ds(0, _SPAN)],
            spanall_ref.at[pl.ds(0, 1)], sem).wait()
        return carry

    lax.fori_loop(0, n_ch, drain, 0)

    zero16 = jnp.zeros((16,), jnp.int32)
    zf16 = jnp.zeros((16,), jnp.float32)

    def u_body(u, carry):
        idxv = rowidx_ref[pl.ds(u * 16, 16)]
        jvec = iota + u * 16
        m = jnp.logical_and(jvec < n_keep, idxv < _SPAN)

        def ci_body(ci, carry2):
            g = plsc.load_gather(
                spanall_ref, [lax.broadcast(ci, (16,)), idxv], mask=m)
            outall_ref[ci, pl.ds(u * 16, 16)] = jnp.where(m, g, zf16)
            return carry2

        lax.fori_loop(0, n_ch, ci_body, 0)
        return carry

    lax.fori_loop(0, _OV, u_body, 0)

    @pl.when(idx_max >= _SPAN)
    def _full_rows():
        def ci_body(ci, carry):
            crow = (ch_lo + ci) * _B + b
            pltpu.async_copy(
                pred_hbm.at[pl.ds(crow, 1)], rowbuf_ref, sem2).wait()

            def jv2(u, carry2):
                idxv = rowidx_ref[pl.ds(u * 16, 16)]
                jvec = iota + u * 16
                m = jnp.logical_and(jvec < n_keep, idxv >= _SPAN)
                g = plsc.load_gather(rowbuf_ref, [zero16, idxv], mask=m)
                cur = outall_ref[ci, pl.ds(u * 16, 16)]
                outall_ref[ci, pl.ds(u * 16, 16)] = jnp.where(m, g, cur)
                return carry2

            lax.fori_loop(0, _OV, jv2, 0)
            return carry

        lax.fori_loop(0, n_ch, ci_body, 0)

    def wb(ci, carry):
        pltpu.async_copy(
            outall_ref.at[pl.ds(ci, 1)],
            out_hbm.at[pl.ds((ch_lo + ci) * _B + b, 1)], osem)
        return carry

    lax.fori_loop(0, n_ch, wb, 0)

    def odrain(ci, carry):
        pltpu.make_async_copy(
            out_hbm.at[pl.ds(0, 1)], outall_ref.at[pl.ds(0, 1)], osem).wait()
        return carry

    lax.fori_loop(0, n_ch, odrain, 0)


@jax.jit
def kernel(prediction):
    pred_t = jnp.transpose(prediction, (2, 0, 1)).reshape(_C * _B, _N)
    mesh = plsc.VectorSubcoreMesh(core_axis_name="c", subcore_axis_name="s")
    out2d = pl.kernel(
        _sc_kernel,
        out_type=jax.ShapeDtypeStruct((_C * _B, _OPAD), jnp.float32),
        mesh=mesh,
        compiler_params=pltpu.CompilerParams(
            needs_layout_passes=False, use_tc_tiling_on_sc=True
        ),
        scratch_types=[
            pltpu.VMEM((1, _N), jnp.float32),
            pltpu.VMEM((_SEL,), jnp.int32),
            pltpu.VMEM((_CH0, _SPAN), jnp.float32),
            pltpu.VMEM((_CH0, _OPAD), jnp.float32),
            pltpu.SemaphoreType.DMA,
            pltpu.SemaphoreType.DMA,
            pltpu.SemaphoreType.DMA,
        ],
    )(pred_t)
    out_t = out2d.reshape(_C, _B, _OPAD)
    return jnp.transpose(out_t, (1, 2, 0))[:, :_MAX_DET, :]


# speedup vs baseline: 29.2183x; 1.0024x over previous
import jax
import jax.numpy as jnp
from jax import lax
from jax.experimental import pallas as pl
from jax.experimental.pallas import tpu as pltpu
from jax.experimental.pallas import tpu_sc as plsc

_B, _N, _C = 16, 20000, 85
_CONF = 0.25
_MAX_DET = 300
_OPAD = 304
_OV = _OPAD // 16
_SEL = 320
_GRP = 1024
_VPG = _GRP // 16
_SCAN0 = 4096
_NGRP = _N // _GRP
_TAIL0 = _NGRP * _GRP
_TAILV = (_N - _TAIL0) // 16
_SPAN = 1024
_NC = 2
_CH0 = 43
_OBJ_ROW = 4 * _B


def _sc_kernel(pred_hbm, out_hbm, rowbuf_ref, rowidx_ref, spanall_ref,
               outall_ref, sem, sem2, osem):
    wid = lax.axis_index("s") * _NC + lax.axis_index("c")
    b = wid >> 1
    half = wid & 1
    ch_lo = half * _CH0
    n_ch = _CH0 - half
    iota = lax.iota(jnp.int32, 16)

    pltpu.async_copy(
        pred_hbm.at[pl.ds(_OBJ_ROW + b, 1), pl.ds(0, _SCAN0)],
        rowbuf_ref.at[pl.ds(0, 1), pl.ds(0, _SCAN0)], sem2)

    def fire(ci, carry):
        pltpu.async_copy(
            pred_hbm.at[pl.ds((ch_lo + ci) * _B + b, 1), pl.ds(0, _SPAN)],
            spanall_ref.at[pl.ds(ci, 1)], sem)
        return carry

    lax.fori_loop(0, n_ch, fire, 0)

    pltpu.make_async_copy(
        pred_hbm.at[pl.ds(0, 1), pl.ds(0, _SCAN0)],
        rowbuf_ref.at[pl.ds(0, 1), pl.ds(0, _SCAN0)], sem2).wait()

    def _init(k, carry):
        rowidx_ref[pl.ds(k * 16, 16)] = jnp.zeros((16,), jnp.int32)
        return carry

    lax.fori_loop(0, _SEL // 16, _init, 0)

    def compact(base, nvec, cnt0):
        def vec_body(j, cnt2):
            v = rowbuf_ref[0, pl.ds(base + j * 16, 16)]
            m = v > _CONF
            plsc.store_compressed(
                rowidx_ref.at[pl.ds(jnp.minimum(cnt2, _MAX_DET), 16)],
                iota + (base + j * 16), mask=m)
            return cnt2 + plsc.all_reduce_population_count(m)[0]

        return lax.fori_loop(0, nvec, vec_body, cnt0)

    def grp(t, cnt):
        return lax.cond(
            cnt < _MAX_DET, lambda: compact(t * _GRP, _VPG, cnt), lambda: cnt)

    cnt = lax.fori_loop(0, _SCAN0 // _GRP, grp, 0)

    def scan_rest():
        pltpu.sync_copy(pred_hbm.at[pl.ds(_OBJ_ROW + b, 1)], rowbuf_ref)
        c2 = lax.fori_loop(_SCAN0 // _GRP, _NGRP, grp, cnt)
        return lax.cond(
            c2 < _MAX_DET, lambda: compact(_TAIL0, _TAILV, c2), lambda: c2)

    cnt = lax.cond(cnt < _MAX_DET, scan_rest, lambda: cnt)
    n_keep = jnp.minimum(cnt, _MAX_DET)
    last = jnp.maximum(n_keep - 1, 0)
    idx_max = rowidx_ref[pl.ds(last, 16)][0]

    def drain(ci, carry):
        pltpu.make_async_copy(
            pred_hbm.at[pl.ds(0, 1), pl.ds(0, _SPAN)],
            spanall_ref.at[pl.ds(0, 1)], sem).wait()
        return carry

    lax.fori_loop(0, n_ch, drain, 0)

    zero16 = jnp.zeros((16,), jnp.int32)
    zf16 = jnp.zeros((16,), jnp.float32)

    def u_body(u, carry):
        idxv = rowidx_ref[pl.ds(u * 16, 16)]
        jvec = iota + u * 16
        m = jnp.logical_and(jvec < n_keep, idxv < _SPAN)

        def ci_body(ci, carry2):
            g = plsc.load_gather(
                spanall_ref, [lax.broadcast(ci, (16,)), idxv], mask=m)
            outall_ref[ci, pl.ds(u * 16, 16)] = jnp.where(m, g, zf16)
            return carry2

        lax.fori_loop(0, n_ch, ci_body, 0)
        return carry

    lax.fori_loop(0, _OV, u_body, 0)

    @pl.when(idx_max >= _SPAN)
    def _full_rows():
        def ci_body(ci, carry):
            crow = (ch_lo + ci) * _B + b
            pltpu.async_copy(
                pred_hbm.at[pl.ds(crow, 1)], rowbuf_ref, sem2).wait()

            def jv2(u, carry2):
                idxv = rowidx_ref[pl.ds(u * 16, 16)]
                jvec = iota + u * 16
                m = jnp.logical_and(jvec < n_keep, idxv >= _SPAN)
                g = plsc.load_gather(rowbuf_ref, [zero16, idxv], mask=m)
                cur = outall_ref[ci, pl.ds(u * 16, 16)]
                outall_ref[ci, pl.ds(u * 16, 16)] = jnp.where(m, g, cur)
                return carry2

            lax.fori_loop(0, _OV, jv2, 0)
            return carry

        lax.fori_loop(0, n_ch, ci_body, 0)

    def wb(ci, carry):
        pltpu.async_copy(
            outall_ref.at[pl.ds(ci, 1)],
            out_hbm.at[pl.ds((ch_lo + ci) * _B + b, 1)], osem)
        return carry

    lax.fori_loop(0, n_ch, wb, 0)

    def odrain(ci, carry):
        pltpu.make_async_copy(
            out_hbm.at[pl.ds(0, 1)], outall_ref.at[pl.ds(0, 1)], osem).wait()
        return carry

    lax.fori_loop(0, n_ch, odrain, 0)


@jax.jit
def kernel(prediction):
    pred_t = jnp.transpose(prediction, (2, 0, 1)).reshape(_C * _B, _N)
    mesh = plsc.VectorSubcoreMesh(core_axis_name="c", subcore_axis_name="s")
    out2d = pl.kernel(
        _sc_kernel,
        out_type=jax.ShapeDtypeStruct((_C * _B, _OPAD), jnp.float32),
        mesh=mesh,
        compiler_params=pltpu.CompilerParams(
            needs_layout_passes=False, use_tc_tiling_on_sc=True
        ),
        scratch_types=[
            pltpu.VMEM((1, _N), jnp.float32),
            pltpu.VMEM((_SEL,), jnp.int32),
            pltpu.VMEM((_CH0, _SPAN), jnp.float32),
            pltpu.VMEM((_CH0, _OPAD), jnp.float32),
            pltpu.SemaphoreType.DMA,
            pltpu.SemaphoreType.DMA,
            pltpu.SemaphoreType.DMA,
        ],
    )(pred_t)
    out_t = out2d.reshape(_C, _B, _OPAD)
    return jnp.transpose(out_t, (1, 2, 0))[:, :_MAX_DET, :]
